# baseline (speedup 1.0000x reference)
import numpy as np

import jax
import jax.numpy as jnp
from jax import lax
from jax.experimental import pallas as pl
from jax.experimental.pallas import tpu as pltpu
from jax.experimental.pallas import tpu_sc as plsc

D0, D1 = 192, 160
OUT0, OUT1 = 64, 32
FH, FW = 3, 5
L = 16
NUM_BINS = 32
NC, NS = 2, 16
GROUPS = 4
HSLOT = NUM_BINS * L

_BIAS = np.tile(np.repeat(np.arange(NUM_BINS - 1, -1, -1), L),
                GROUPS).astype(np.float32)


def _body(xx_hbm, bias_hbm, out_hbm, buf, hist, outbuf, sem1, sem2):
    wid = lax.axis_index("s") * NC + lax.axis_index("c")
    n_in = 2 * FH * D1
    cin = pltpu.async_copy(xx_hbm.at[pl.ds(wid * n_in, n_in)], buf, sem1)
    cbias = pltpu.async_copy(bias_hbm, hist, sem2)
    cin.wait()
    cbias.wait()

    lanes = lax.iota(jnp.int32, L)
    col0 = lanes * FW
    hit = jnp.full((L,), float(NUM_BINS), jnp.float32)
    lane_bias = lanes + L * L

    def group(g, carry):
        base = (g >> 1) * (FH * D1) + (g & 1) * (L * FW)
        gbase = g * HSLOT
        basev = col0 + base
        idx0 = lane_bias + gbase
        for p in range(FH * FW):
            off = (p // FW) * D1 + (p % FW)
            x = plsc.load_gather(buf, [basev + off])
            y = jnp.minimum(jnp.maximum(x * 8.0, -16.0), 15.0)
            iy = y.astype(jnp.int32)
            q = iy + (iy.astype(jnp.float32) < y).astype(jnp.int32)
            plsc.addupdate_scatter(hist, [(q << 4) + idx0], hit)
        best = hist[pl.ds(gbase, L)]
        for b in range(1, NUM_BINS):
            best = jnp.maximum(best, hist[pl.ds(gbase + L * b, L)])
        mode = (NUM_BINS - 1) - (best.astype(jnp.int32) & (NUM_BINS - 1))
        outbuf[pl.ds(g * L, L)] = mode.astype(jnp.float32)
        return carry

    lax.fori_loop(0, GROUPS, group, 0)
    pltpu.sync_copy(outbuf, out_hbm.at[pl.ds(wid * 2 * OUT1, 2 * OUT1)])


_hist_call = pl.kernel(
    _body,
    out_type=jax.ShapeDtypeStruct((OUT0 * OUT1,), jnp.float32),
    mesh=plsc.VectorSubcoreMesh(core_axis_name="c", subcore_axis_name="s"),
    compiler_params=pltpu.CompilerParams(needs_layout_passes=False),
    scratch_types=[
        pltpu.VMEM((2 * FH * D1,), jnp.float32),
        pltpu.VMEM((GROUPS * HSLOT,), jnp.float32),
        pltpu.VMEM((2 * OUT1,), jnp.float32),
        pltpu.SemaphoreType.DMA,
        pltpu.SemaphoreType.DMA,
    ],
)


@jax.jit
def kernel(xx):
    return _hist_call(xx.reshape(D0 * D1), jnp.asarray(_BIAS)).reshape(OUT0, OUT1)


# speedup vs baseline: 1.0481x; 1.0481x over previous
import jax
import jax.numpy as jnp
from jax import lax
from jax.experimental import pallas as pl
from jax.experimental.pallas import tpu as pltpu
from jax.experimental.pallas import tpu_sc as plsc

D0, D1 = 192, 160
OUT0, OUT1 = 64, 32
FH, FW = 3, 5
L = 16
NUM_BINS = 32
NC, NS = 2, 16
GROUPS = 4
HSLOT = NUM_BINS * L


def _body(xx_hbm, out_hbm, buf, hist, outbuf):
    wid = lax.axis_index("s") * NC + lax.axis_index("c")
    n_in = 2 * FH * D1
    pltpu.sync_copy(xx_hbm.at[pl.ds(wid * n_in, n_in)], buf)

    lanes = lax.iota(jnp.int32, L)
    col0 = lanes * FW
    hit = jnp.full((L,), float(NUM_BINS), jnp.float32)
    lane_bias = lanes + L * L

    def group(g, carry):
        base = (g >> 1) * (FH * D1) + (g & 1) * (L * FW)
        basev = col0 + base
        idx0 = lane_bias
        for b in range(NUM_BINS):
            hist[pl.ds(L * b, L)] = jnp.full((L,), float(NUM_BINS - 1 - b),
                                             jnp.float32)
        for p in range(FH * FW):
            off = (p // FW) * D1 + (p % FW)
            x = plsc.load_gather(buf, [basev + off])
            y = jnp.minimum(jnp.maximum(x * 8.0, -16.0), 15.0)
            iy = y.astype(jnp.int32)
            q = iy + (iy.astype(jnp.float32) < y).astype(jnp.int32)
            plsc.addupdate_scatter(hist, [(q << 4) + idx0], hit)
        best = hist[pl.ds(0, L)]
        for b in range(1, NUM_BINS):
            best = jnp.maximum(best, hist[pl.ds(L * b, L)])
        mode = (NUM_BINS - 1) - (best.astype(jnp.int32) & (NUM_BINS - 1))
        outbuf[pl.ds(g * L, L)] = mode.astype(jnp.float32)
        return carry

    lax.fori_loop(0, GROUPS, group, 0)
    pltpu.sync_copy(outbuf, out_hbm.at[pl.ds(wid * 2 * OUT1, 2 * OUT1)])


_hist_call = pl.kernel(
    _body,
    out_type=jax.ShapeDtypeStruct((OUT0 * OUT1,), jnp.float32),
    mesh=plsc.VectorSubcoreMesh(core_axis_name="c", subcore_axis_name="s"),
    compiler_params=pltpu.CompilerParams(needs_layout_passes=False),
    scratch_types=[
        pltpu.VMEM((2 * FH * D1,), jnp.float32),
        pltpu.VMEM((HSLOT,), jnp.float32),
        pltpu.VMEM((2 * OUT1,), jnp.float32),
    ],
)


@jax.jit
def kernel(xx):
    return _hist_call(xx.reshape(D0 * D1)).reshape(OUT0, OUT1)
